# Initial kernel scaffold; baseline (speedup 1.0000x reference)
import jax
import jax.numpy as jnp
from jax.experimental import pallas as pl


def kernel(ob_no, counts, obs_low, obs_high):
    raise NotImplementedError("write your pallas kernel here")


# speedup vs baseline: 9.1757x; 9.1757x over previous
import functools

import jax
import jax.numpy as jnp
from jax import lax
from jax.experimental import pallas as pl
from jax.experimental.pallas import tpu as pltpu
from jax.experimental.pallas import tpu_sc as plsc

CBE = 0.1
L = 16
NC = 2
NS = 16
NW = NC * NS


def _table_body(counts_ref, out_ref):
    out_ref[...] = CBE * lax.rsqrt(counts_ref[...] + 1.0)


def _make_sc_kernel(n, n_bins2, n_bins_minor, per_w):
    mesh = plsc.VectorSubcoreMesh(core_axis_name="c", subcore_axis_name="s")

    @functools.partial(
        pl.kernel,
        mesh=mesh,
        out_type=jax.ShapeDtypeStruct((n,), jnp.float32),
        compiler_params=pltpu.CompilerParams(needs_layout_passes=False),
        scratch_types=[
            pltpu.VMEM((n_bins2,), jnp.float32),
            pltpu.VMEM((per_w * 2,), jnp.float32),
            pltpu.VMEM((per_w,), jnp.float32),
            pltpu.VMEM((4 * L,), jnp.float32),
        ],
    )
    def sc_kernel(table_hbm, obs_hbm, params_hbm, out_hbm,
                  table_v, obs_v, out_v, params_v):
        wid = lax.axis_index("s") * NC + lax.axis_index("c")
        base = wid * per_w
        pltpu.sync_copy(table_hbm, table_v)
        pltpu.sync_copy(params_hbm, params_v)
        pltpu.sync_copy(obs_hbm.at[pl.ds(base * 2, per_w * 2)], obs_v)

        scale_x = params_v[pl.ds(0 * L, L)]
        scale_y = params_v[pl.ds(1 * L, L)]
        hi_x = params_v[pl.ds(2 * L, L)]
        hi_y = params_v[pl.ds(3 * L, L)]
        iota2 = lax.iota(jnp.int32, L) * 2

        def body(j, carry):
            i0 = j * (2 * L) + iota2
            x = plsc.load_gather(obs_v, [i0])
            y = plsc.load_gather(obs_v, [i0 + 1])
            sx = x * scale_x
            sy = y * scale_y
            sx = jnp.where(sx >= hi_x, sx - 1.0, sx)
            sy = jnp.where(sy >= hi_y, sy - 1.0, sy)
            idx = sx.astype(jnp.int32) * n_bins_minor + sy.astype(jnp.int32)
            out_v[pl.ds(j * L, L)] = plsc.load_gather(table_v, [idx])
            return carry

        lax.fori_loop(0, per_w // L, body, 0)
        pltpu.sync_copy(out_v, out_hbm.at[pl.ds(base, per_w)])

    return sc_kernel


def kernel(ob_no, counts, obs_low, obs_high):
    n, obs_dim = ob_no.shape
    b0, b1 = counts.shape
    assert obs_dim == 2 and b0 == b1
    assert n % (NW * L) == 0
    per_w = n // NW

    bonus = pl.pallas_call(
        _table_body,
        out_shape=jax.ShapeDtypeStruct((b0, b1), jnp.float32),
    )(counts)

    scale = obs_high - obs_low
    params = jnp.concatenate([
        jnp.full((L,), scale[0], jnp.float32),
        jnp.full((L,), scale[1], jnp.float32),
        jnp.full((L,), obs_high[0], jnp.float32),
        jnp.full((L,), obs_high[1], jnp.float32),
    ])

    sc = _make_sc_kernel(n, b0 * b1, b1, per_w)
    return sc(bonus.reshape(-1), ob_no.reshape(-1), params)


# speedup vs baseline: 9.3590x; 1.0200x over previous
import functools

import jax
import jax.numpy as jnp
from jax import lax
from jax.experimental import pallas as pl
from jax.experimental.pallas import tpu as pltpu
from jax.experimental.pallas import tpu_sc as plsc

CBE = 0.1
L = 16
NC = 2
NS = 16
NW = NC * NS


def _table_body(counts_ref, out_ref):
    out_ref[...] = CBE * lax.rsqrt(counts_ref[...] + 1.0)


def _make_sc_kernel(n, n_bins2, n_bins_minor, per_w):
    mesh = plsc.VectorSubcoreMesh(core_axis_name="c", subcore_axis_name="s")

    @functools.partial(
        pl.kernel,
        mesh=mesh,
        out_type=jax.ShapeDtypeStruct((n,), jnp.float32),
        compiler_params=pltpu.CompilerParams(needs_layout_passes=False),
        scratch_types=[
            pltpu.VMEM((n_bins2,), jnp.float32),
            pltpu.VMEM((per_w * 2,), jnp.float32),
            pltpu.VMEM((per_w,), jnp.float32),
            pltpu.VMEM((4 * L,), jnp.float32),
        ],
    )
    def sc_kernel(table_hbm, obs_hbm, params_hbm, out_hbm,
                  table_v, obs_v, out_v, params_v):
        wid = lax.axis_index("s") * NC + lax.axis_index("c")
        base = wid * per_w
        pltpu.sync_copy(table_hbm, table_v)
        pltpu.sync_copy(params_hbm, params_v)
        pltpu.sync_copy(obs_hbm.at[pl.ds(base * 2, per_w * 2)], obs_v)

        scale_x = params_v[pl.ds(0 * L, L)]
        scale_y = params_v[pl.ds(1 * L, L)]
        hi_x = params_v[pl.ds(2 * L, L)]
        hi_y = params_v[pl.ds(3 * L, L)]
        iota2 = lax.iota(jnp.int32, L) * 2

        @plsc.parallel_loop(0, per_w // L, unroll=16)
        def body(j):
            i0 = j * (2 * L) + iota2
            x = plsc.load_gather(obs_v, [i0])
            y = plsc.load_gather(obs_v, [i0 + 1])
            sx = x * scale_x
            sy = y * scale_y
            sx = jnp.where(sx >= hi_x, sx - 1.0, sx)
            sy = jnp.where(sy >= hi_y, sy - 1.0, sy)
            idx = sx.astype(jnp.int32) * n_bins_minor + sy.astype(jnp.int32)
            out_v[pl.ds(j * L, L)] = plsc.load_gather(table_v, [idx])
        pltpu.sync_copy(out_v, out_hbm.at[pl.ds(base, per_w)])

    return sc_kernel


def kernel(ob_no, counts, obs_low, obs_high):
    n, obs_dim = ob_no.shape
    b0, b1 = counts.shape
    assert obs_dim == 2 and b0 == b1
    assert n % (NW * L) == 0
    per_w = n // NW

    bonus = pl.pallas_call(
        _table_body,
        out_shape=jax.ShapeDtypeStruct((b0, b1), jnp.float32),
    )(counts)

    scale = obs_high - obs_low
    params = jnp.concatenate([
        jnp.full((L,), scale[0], jnp.float32),
        jnp.full((L,), scale[1], jnp.float32),
        jnp.full((L,), obs_high[0], jnp.float32),
        jnp.full((L,), obs_high[1], jnp.float32),
    ])

    sc = _make_sc_kernel(n, b0 * b1, b1, per_w)
    return sc(bonus.reshape(-1), ob_no.reshape(-1), params)
